# baseline (speedup 1.0000x reference)
import jax
import jax.numpy as jnp
from jax.experimental import pallas as pl

K = 1024
DIM = 32
HW = 1024
B = 32
BB = 2
COMMITMENT_COST = 0.25


def _vq_body(z_ref, emb_ref, zq_ref, loss_ref):
    step = pl.program_id(0)
    emb = emb_ref[...]
    emb_m2 = emb * -2.0
    e2 = jnp.sum(emb * emb, axis=1)
    part = jnp.zeros((1, 1), jnp.float32)
    for i in range(BB):
        z = z_ref[i].reshape(DIM, HW)
        x2 = jnp.sum(z * z, axis=0)
        dt = jax.lax.dot_general(
            z, emb_m2, (((0,), (1,)), ((), ())),
            preferred_element_type=jnp.float32)
        d = (x2[:, None] + e2[None, :]) + dt
        minv = jnp.min(d, axis=1)
        ciota = jax.lax.broadcasted_iota(jnp.int32, (HW, K), 1)
        is_min = d == minv[:, None]
        idx = jnp.min(jnp.where(is_min, ciota, K), axis=1)
        onehot = (idx[:, None] == ciota).astype(jnp.bfloat16)
        zq = jax.lax.dot_general(
            emb, onehot, (((0,), (1,)), ((), ())),
            preferred_element_type=jnp.float32)
        zq_ref[i] = (z + (zq - z)).reshape(DIM, 32, 32)
        part = part + jnp.sum(minv).reshape(1, 1)

    @pl.when(step == 0)
    def _():
        loss_ref[...] = part

    @pl.when(step != 0)
    def _():
        loss_ref[...] += part


def kernel(z_e, emb_weight):
    z_q_st, loss_raw = pl.pallas_call(
        _vq_body,
        grid=(B // BB,),
        in_specs=[
            pl.BlockSpec((BB, DIM, 32, 32), lambda b: (b, 0, 0, 0)),
            pl.BlockSpec((K, DIM), lambda b: (0, 0)),
        ],
        out_specs=[
            pl.BlockSpec((BB, DIM, 32, 32), lambda b: (b, 0, 0, 0)),
            pl.BlockSpec((1, 1), lambda b: (0, 0)),
        ],
        out_shape=[
            jax.ShapeDtypeStruct((B, DIM, 32, 32), jnp.float32),
            jax.ShapeDtypeStruct((1, 1), jnp.float32),
        ],
    )(z_e, emb_weight)
    loss = loss_raw[0, 0] * ((1.0 + COMMITMENT_COST) / (B * DIM * HW))
    return (z_q_st, loss)
